# baseline (speedup 1.0000x reference)
import functools

import jax
import jax.numpy as jnp
from jax import lax
from jax.experimental import pallas as pl
from jax.experimental.pallas import tpu as pltpu
from jax.experimental.pallas import tpu_sc as plsc

N = 4096
B = 1024

NC = 2
NS = 16
L = 16

PASSES = 5
RP_BIG = 440
RP_LAST = N // NC - 4 * RP_BIG
R = RP_BIG * N
RS_BIG = R // NS
RS_LAST = RP_LAST * N // NS

CH = 2048
GW = 128
GROUPS = CH // GW


def _scatter_body(flat_hbm, vals_hbm, bidx_hbm, bvals_hbm,
                  zeros_hbm, w_hbm, bias_hbm,
                  sp, bias_sp, fbuf, vbuf, ibuf, obuf,
                  lsem0, lsem1, ssem0, ssem1):
    c = lax.axis_index("c")
    s = lax.axis_index("s")
    nnzp = flat_hbm.shape[0] - CH
    chunks = nnzp // (NS * CH)

    lane = lax.iota(jnp.int32, L)
    trash = R + s * L + lane
    lsems = (lsem0, lsem1)
    ssems = (ssem0, ssem1)

    def fire_loads(ch, slot):
        off = (s * chunks + ch) * CH
        pltpu.async_copy(flat_hbm.at[pl.ds(off, CH)], fbuf.at[slot],
                         lsems[slot])
        pltpu.async_copy(vals_hbm.at[pl.ds(off, CH)], vbuf.at[slot],
                         lsems[slot])

    def wait_loads(ch, slot):
        off = (s * chunks + ch) * CH
        pltpu.make_async_copy(flat_hbm.at[pl.ds(off, CH)], fbuf.at[slot],
                              lsems[slot]).wait()
        pltpu.make_async_copy(vals_hbm.at[pl.ds(off, CH)], vbuf.at[slot],
                              lsems[slot]).wait()

    def drain_slot(slot):
        for rr in range(GROUPS):
            pltpu.make_async_copy(obuf.at[slot, rr],
                                  sp.at[ibuf.at[slot, rr]],
                                  ssems[slot]).wait()

    def do_chunk(slot, base_n, m):
        for rr in range(GROUPS):
            for jj in range(GW // L):
                sl = pl.ds(rr * GW + jj * L, L)
                local = fbuf[slot, sl] - base_n
                v = vbuf[slot, sl]
                inr = local.astype(jnp.uint32) < m.astype(jnp.uint32)
                ibuf[slot, rr, pl.ds(jj * L, L)] = jnp.where(inr, local,
                                                             trash)
                obuf[slot, rr, pl.ds(jj * L, L)] = jnp.where(inr, v, 0.0)
        for rr in range(GROUPS):
            pltpu.async_copy(obuf.at[slot, rr], sp.at[ibuf.at[slot, rr]],
                             ssems[slot], add=True)

    def do_pass(p, _):
        base = (c * (N // NC) + p * RP_BIG) * N
        m = jnp.where(p < 4, RP_BIG * N, RP_LAST * N)

        @pl.when(p < 4)
        def _():
            pltpu.sync_copy(zeros_hbm, sp.at[pl.ds(s * RS_BIG, RS_BIG)])

        @pl.when(p == 4)
        def _():
            pltpu.sync_copy(zeros_hbm.at[pl.ds(0, RS_LAST)],
                            sp.at[pl.ds(s * RS_LAST, RS_LAST)])

        plsc.subcore_barrier()
        fire_loads(0, 0)

        def chunk_pair(ch2, _):
            ch = 2 * ch2
            fire_loads(ch + 1, 1)
            wait_loads(ch, 0)

            @pl.when(ch2 > 0)
            def _():
                drain_slot(0)

            do_chunk(0, base, m)
            fire_loads(ch + 2, 0)
            wait_loads(ch + 1, 1)

            @pl.when(ch2 > 0)
            def _():
                drain_slot(1)

            do_chunk(1, base, m)
            return 0

        lax.fori_loop(0, chunks // 2, chunk_pair, 0)
        drain_slot(0)
        drain_slot(1)
        wait_loads(chunks, 0)
        plsc.subcore_barrier()

        @pl.when(p < 4)
        def _():
            pltpu.sync_copy(sp.at[pl.ds(s * RS_BIG, RS_BIG)],
                            w_hbm.at[pl.ds(base + s * RS_BIG, RS_BIG)])

        @pl.when(p == 4)
        def _():
            pltpu.sync_copy(sp.at[pl.ds(s * RS_LAST, RS_LAST)],
                            w_hbm.at[pl.ds(base + s * RS_LAST, RS_LAST)])

        plsc.subcore_barrier()
        return 0

    lax.fori_loop(0, PASSES, do_pass, 0)

    @pl.when((c == 0) & (s == 0))
    def _():
        pltpu.sync_copy(zeros_hbm.at[pl.ds(0, N)], bias_sp)
        pltpu.sync_copy(bidx_hbm, fbuf.at[0])
        pltpu.sync_copy(bvals_hbm, vbuf.at[0])
        for rr in range(GROUPS):
            for jj in range(GW // L):
                sl = pl.ds(rr * GW + jj * L, L)
                ibuf[0, rr, pl.ds(jj * L, L)] = fbuf[0, sl]
                obuf[0, rr, pl.ds(jj * L, L)] = vbuf[0, sl]
            pltpu.async_copy(obuf.at[0, rr], bias_sp.at[ibuf.at[0, rr]],
                             ssem0, add=True).wait()
        pltpu.sync_copy(bias_sp, bias_hbm)


def _build_w_bias(rows, cols, vals, bidx, bvals):
    nnz = rows.shape[0]
    grp = NS * CH * 2
    nnzp = ((nnz + grp - 1) // grp) * grp
    pad = nnzp + CH - nnz
    flat = rows * N + cols
    flat = jnp.concatenate([flat, jnp.full((pad,), 1 << 30, jnp.int32)])
    vals = jnp.concatenate([vals, jnp.zeros((pad,), jnp.float32)])
    zeros = jnp.zeros((RS_BIG,), jnp.float32)

    mesh = plsc.VectorSubcoreMesh(core_axis_name="c", subcore_axis_name="s")
    f = pl.kernel(
        _scatter_body,
        out_type=[jax.ShapeDtypeStruct((N * N,), jnp.float32),
                  jax.ShapeDtypeStruct((N,), jnp.float32)],
        mesh=mesh,
        scratch_types=[
            pltpu.VMEM_SHARED((R + NS * L,), jnp.float32),
            pltpu.VMEM_SHARED((N,), jnp.float32),
            pltpu.VMEM((2, CH), jnp.int32),
            pltpu.VMEM((2, CH), jnp.float32),
            pltpu.VMEM((2, GROUPS, GW), jnp.int32),
            pltpu.VMEM((2, GROUPS, GW), jnp.float32),
            pltpu.SemaphoreType.DMA,
            pltpu.SemaphoreType.DMA,
            pltpu.SemaphoreType.DMA,
            pltpu.SemaphoreType.DMA,
        ],
    )
    return f(flat, vals, bidx, bvals, zeros)


BN = 256
BK = 1024


def _matmul_body(x_ref, w_ref, b_ref, o_ref):
    k = pl.program_id(1)

    @pl.when(k == 0)
    def _():
        o_ref[...] = jnp.broadcast_to(b_ref[...], o_ref.shape)

    o_ref[...] += lax.dot_general(
        x_ref[...], w_ref[...],
        (((1,), (1,)), ((), ())),
        preferred_element_type=jnp.float32,
        precision=lax.Precision.DEFAULT,
    )


def _matmul(x, w, bias):
    grid = (N // BN, N // BK)
    return pl.pallas_call(
        _matmul_body,
        grid=grid,
        in_specs=[
            pl.BlockSpec((B, BK), lambda n, k: (0, k)),
            pl.BlockSpec((BN, BK), lambda n, k: (n, k)),
            pl.BlockSpec((1, BN), lambda n, k: (0, n)),
        ],
        out_specs=pl.BlockSpec((B, BN), lambda n, k: (0, n)),
        out_shape=jax.ShapeDtypeStruct((B, N), jnp.float32),
        compiler_params=pltpu.CompilerParams(
            dimension_semantics=("parallel", "arbitrary")),
    )(x, w, bias)


@jax.jit
def kernel(input, weight_rows, weight_cols, weight_vals, bias_idx, bias_vals):
    w_flat, bias = _build_w_bias(weight_rows.astype(jnp.int32),
                                 weight_cols.astype(jnp.int32),
                                 weight_vals, bias_idx.astype(jnp.int32),
                                 bias_vals)
    w = w_flat.reshape(N, N)
    return _matmul(input, w, bias.reshape(1, N))


# speedup vs baseline: 1.1140x; 1.1140x over previous
import functools

import jax
import jax.numpy as jnp
from jax import lax
from jax.experimental import pallas as pl
from jax.experimental.pallas import tpu as pltpu
from jax.experimental.pallas import tpu_sc as plsc

N = 4096
B = 1024

NC = 2
NS = 16
L = 16

PASSES = 5
RP_BIG = 440
RP_LAST = N // NC - 4 * RP_BIG
R = RP_BIG * N
RS_BIG = R // NS
RS_LAST = RP_LAST * N // NS

CH = 2048
GW = 128
GROUPS = CH // GW


def _scatter_body(flat_hbm, vals_hbm, bidx_hbm, bvals_hbm,
                  zeros_hbm, w_hbm, bias_hbm,
                  sp, bias_sp, fbuf, vbuf, ibuf, obuf,
                  lsem0, lsem1, ssem0, ssem1):
    c = lax.axis_index("c")
    s = lax.axis_index("s")
    nnzp = flat_hbm.shape[0] - CH
    chunks = nnzp // (NS * CH)

    lane = lax.iota(jnp.int32, L)
    trash = R + s * L + lane
    lsems = (lsem0, lsem1)
    ssems = (ssem0, ssem1)

    def fire_loads(ch, slot):
        off = (s * chunks + ch) * CH
        pltpu.async_copy(flat_hbm.at[pl.ds(off, CH)], fbuf.at[slot],
                         lsems[slot])
        pltpu.async_copy(vals_hbm.at[pl.ds(off, CH)], vbuf.at[slot],
                         lsems[slot])

    def wait_loads(ch, slot):
        off = (s * chunks + ch) * CH
        pltpu.make_async_copy(flat_hbm.at[pl.ds(off, CH)], fbuf.at[slot],
                              lsems[slot]).wait()
        pltpu.make_async_copy(vals_hbm.at[pl.ds(off, CH)], vbuf.at[slot],
                              lsems[slot]).wait()

    def drain_slot(slot):
        for rr in range(GROUPS):
            pltpu.make_async_copy(obuf.at[slot, rr],
                                  sp.at[ibuf.at[slot, rr]],
                                  ssems[slot]).wait()

    def do_chunk(slot, base_n, m):
        for rr in range(GROUPS):
            for jj in range(GW // L):
                sl = pl.ds(rr * GW + jj * L, L)
                local = fbuf[slot, sl] - base_n
                v = vbuf[slot, sl]
                inr = local.astype(jnp.uint32) < m.astype(jnp.uint32)
                ibuf[slot, rr, pl.ds(jj * L, L)] = jnp.where(inr, local,
                                                             trash)
                obuf[slot, rr, pl.ds(jj * L, L)] = jnp.where(inr, v, 0.0)
        for rr in range(GROUPS):
            pltpu.async_copy(obuf.at[slot, rr], sp.at[ibuf.at[slot, rr]],
                             ssems[slot], add=True)

    def do_pass(p, _):
        base = (c * (N // NC) + p * RP_BIG) * N
        m = jnp.where(p < 4, RP_BIG * N, RP_LAST * N)

        @pl.when(p < 4)
        def _():
            pltpu.sync_copy(zeros_hbm, sp.at[pl.ds(s * RS_BIG, RS_BIG)])

        @pl.when(p == 4)
        def _():
            pltpu.sync_copy(zeros_hbm.at[pl.ds(0, RS_LAST)],
                            sp.at[pl.ds(s * RS_LAST, RS_LAST)])

        plsc.subcore_barrier()
        fire_loads(0, 0)

        def chunk_pair(ch2, _):
            ch = 2 * ch2
            fire_loads(ch + 1, 1)
            wait_loads(ch, 0)

            @pl.when(ch2 > 0)
            def _():
                drain_slot(0)

            do_chunk(0, base, m)
            fire_loads(ch + 2, 0)
            wait_loads(ch + 1, 1)

            @pl.when(ch2 > 0)
            def _():
                drain_slot(1)

            do_chunk(1, base, m)
            return 0

        lax.fori_loop(0, chunks // 2, chunk_pair, 0)
        drain_slot(0)
        drain_slot(1)
        wait_loads(chunks, 0)
        plsc.subcore_barrier()

        @pl.when(p < 4)
        def _():
            pltpu.sync_copy(sp.at[pl.ds(s * RS_BIG, RS_BIG)],
                            w_hbm.at[pl.ds(base + s * RS_BIG, RS_BIG)])

        @pl.when(p == 4)
        def _():
            pltpu.sync_copy(sp.at[pl.ds(s * RS_LAST, RS_LAST)],
                            w_hbm.at[pl.ds(base + s * RS_LAST, RS_LAST)])

        plsc.subcore_barrier()
        return 0

    lax.fori_loop(0, PASSES, do_pass, 0)

    @pl.when((c == 0) & (s == 0))
    def _():
        pltpu.sync_copy(zeros_hbm.at[pl.ds(0, N)], bias_sp)
        pltpu.sync_copy(bidx_hbm, fbuf.at[0])
        pltpu.sync_copy(bvals_hbm, vbuf.at[0])
        for rr in range(GROUPS):
            for jj in range(GW // L):
                sl = pl.ds(rr * GW + jj * L, L)
                ibuf[0, rr, pl.ds(jj * L, L)] = fbuf[0, sl]
                obuf[0, rr, pl.ds(jj * L, L)] = vbuf[0, sl]
            pltpu.async_copy(obuf.at[0, rr], bias_sp.at[ibuf.at[0, rr]],
                             ssem0, add=True).wait()
        pltpu.sync_copy(bias_sp, bias_hbm)


def _build_w_bias(rows, cols, vals, bidx, bvals):
    nnz = rows.shape[0]
    grp = NS * CH * 2
    nnzp = ((nnz + grp - 1) // grp) * grp
    pad = nnzp + CH - nnz
    flat = rows * N + cols
    flat = jnp.concatenate([flat, jnp.full((pad,), 1 << 30, jnp.int32)])
    vals = jnp.concatenate([vals, jnp.zeros((pad,), jnp.float32)])
    zeros = jnp.zeros((RS_BIG,), jnp.float32)

    mesh = plsc.VectorSubcoreMesh(core_axis_name="c", subcore_axis_name="s")
    f = pl.kernel(
        _scatter_body,
        out_type=[jax.ShapeDtypeStruct((N * N,), jnp.float32),
                  jax.ShapeDtypeStruct((N,), jnp.float32)],
        mesh=mesh,
        scratch_types=[
            pltpu.VMEM_SHARED((R + NS * L,), jnp.float32),
            pltpu.VMEM_SHARED((N,), jnp.float32),
            pltpu.VMEM((2, CH), jnp.int32),
            pltpu.VMEM((2, CH), jnp.float32),
            pltpu.VMEM((2, GROUPS, GW), jnp.int32),
            pltpu.VMEM((2, GROUPS, GW), jnp.float32),
            pltpu.SemaphoreType.DMA,
            pltpu.SemaphoreType.DMA,
            pltpu.SemaphoreType.DMA,
            pltpu.SemaphoreType.DMA,
        ],
    )
    return f(flat, vals, bidx, bvals, zeros)


BN = 512
BK = 2048


def _matmul_body(x_ref, w_ref, b_ref, o_ref):
    k = pl.program_id(1)

    @pl.when(k == 0)
    def _():
        o_ref[...] = jnp.broadcast_to(b_ref[...], o_ref.shape)

    o_ref[...] += lax.dot_general(
        x_ref[...], w_ref[...],
        (((1,), (1,)), ((), ())),
        preferred_element_type=jnp.float32,
        precision=lax.Precision.DEFAULT,
    )


def _matmul(x, w, bias):
    grid = (N // BN, N // BK)
    return pl.pallas_call(
        _matmul_body,
        grid=grid,
        in_specs=[
            pl.BlockSpec((B, BK), lambda n, k: (0, k)),
            pl.BlockSpec((BN, BK), lambda n, k: (n, k)),
            pl.BlockSpec((1, BN), lambda n, k: (0, n)),
        ],
        out_specs=pl.BlockSpec((B, BN), lambda n, k: (0, n)),
        out_shape=jax.ShapeDtypeStruct((B, N), jnp.float32),
        compiler_params=pltpu.CompilerParams(
            dimension_semantics=("parallel", "arbitrary")),
    )(x, w, bias)


@jax.jit
def kernel(input, weight_rows, weight_cols, weight_vals, bias_idx, bias_vals):
    w_flat, bias = _build_w_bias(weight_rows.astype(jnp.int32),
                                 weight_cols.astype(jnp.int32),
                                 weight_vals, bias_idx.astype(jnp.int32),
                                 bias_vals)
    w = w_flat.reshape(N, N)
    return _matmul(input, w, bias.reshape(1, N))


# speedup vs baseline: 1.1606x; 1.0418x over previous
import functools

import jax
import jax.numpy as jnp
from jax import lax
from jax.experimental import pallas as pl
from jax.experimental.pallas import tpu as pltpu
from jax.experimental.pallas import tpu_sc as plsc

N = 4096
B = 1024

NC = 2
NS = 16
L = 16

PASSES = 5
RP_BIG = 440
RP_LAST = N // NC - 4 * RP_BIG
R = RP_BIG * N
RS_BIG = R // NS
RS_LAST = RP_LAST * N // NS

CH = 2048
GW = 128
GROUPS = CH // GW


def _scatter_body(flat_hbm, vals_hbm, bidx_hbm, bvals_hbm,
                  zeros_hbm, w_hbm, bias_hbm,
                  sp, bias_sp, fbuf, vbuf, ibuf, obuf,
                  lsem0, lsem1, ssem0, ssem1):
    c = lax.axis_index("c")
    s = lax.axis_index("s")
    nnzp = flat_hbm.shape[0] - CH
    chunks = nnzp // (NS * CH)

    lane = lax.iota(jnp.int32, L)
    trash = R + s * L + lane
    lsems = (lsem0, lsem1)
    ssems = (ssem0, ssem1)

    def fire_loads(ch, slot):
        off = (s * chunks + ch) * CH
        pltpu.async_copy(flat_hbm.at[pl.ds(off, CH)], fbuf.at[slot],
                         lsems[slot])
        pltpu.async_copy(vals_hbm.at[pl.ds(off, CH)], vbuf.at[slot],
                         lsems[slot])

    def wait_loads(ch, slot):
        off = (s * chunks + ch) * CH
        pltpu.make_async_copy(flat_hbm.at[pl.ds(off, CH)], fbuf.at[slot],
                              lsems[slot]).wait()
        pltpu.make_async_copy(vals_hbm.at[pl.ds(off, CH)], vbuf.at[slot],
                              lsems[slot]).wait()

    def drain_slot(slot):
        for rr in range(GROUPS):
            pltpu.make_async_copy(obuf.at[slot, rr],
                                  sp.at[ibuf.at[slot, rr]],
                                  ssems[slot]).wait()

    def do_chunk(slot, base_n, m):
        for rr in range(GROUPS):
            for jj in range(GW // L):
                sl = pl.ds(rr * GW + jj * L, L)
                local = fbuf[slot, sl] - base_n
                v = vbuf[slot, sl]
                inr = local.astype(jnp.uint32) < m.astype(jnp.uint32)
                ibuf[slot, rr, pl.ds(jj * L, L)] = jnp.where(inr, local,
                                                             trash)
                obuf[slot, rr, pl.ds(jj * L, L)] = jnp.where(inr, v, 0.0)
        for rr in range(GROUPS):
            pltpu.async_copy(obuf.at[slot, rr], sp.at[ibuf.at[slot, rr]],
                             ssems[slot], add=True)

    def do_pass(p, _):
        base = (c * (N // NC) + p * RP_BIG) * N
        m = jnp.where(p < 4, RP_BIG * N, RP_LAST * N)

        @pl.when(p < 4)
        def _():
            pltpu.sync_copy(zeros_hbm, sp.at[pl.ds(s * RS_BIG, RS_BIG)])

        @pl.when(p == 4)
        def _():
            pltpu.sync_copy(zeros_hbm.at[pl.ds(0, RS_LAST)],
                            sp.at[pl.ds(s * RS_LAST, RS_LAST)])

        plsc.subcore_barrier()
        fire_loads(0, 0)

        def chunk_pair(ch2, _):
            ch = 2 * ch2
            fire_loads(ch + 1, 1)
            wait_loads(ch, 0)

            @pl.when(ch2 > 0)
            def _():
                drain_slot(0)

            do_chunk(0, base, m)
            fire_loads(ch + 2, 0)
            wait_loads(ch + 1, 1)

            @pl.when(ch2 > 0)
            def _():
                drain_slot(1)

            do_chunk(1, base, m)
            return 0

        lax.fori_loop(0, chunks // 2, chunk_pair, 0)
        drain_slot(0)
        drain_slot(1)
        wait_loads(chunks, 0)
        plsc.subcore_barrier()

        @pl.when(p < 4)
        def _():
            pltpu.sync_copy(sp.at[pl.ds(s * RS_BIG, RS_BIG)],
                            w_hbm.at[pl.ds(base + s * RS_BIG, RS_BIG)])

        @pl.when(p == 4)
        def _():
            pltpu.sync_copy(sp.at[pl.ds(s * RS_LAST, RS_LAST)],
                            w_hbm.at[pl.ds(base + s * RS_LAST, RS_LAST)])

        plsc.subcore_barrier()
        return 0

    lax.fori_loop(0, PASSES, do_pass, 0)

    @pl.when((c == 0) & (s == 0))
    def _():
        pltpu.sync_copy(zeros_hbm.at[pl.ds(0, N)], bias_sp)
        pltpu.sync_copy(bidx_hbm, fbuf.at[0])
        pltpu.sync_copy(bvals_hbm, vbuf.at[0])
        for rr in range(GROUPS):
            for jj in range(GW // L):
                sl = pl.ds(rr * GW + jj * L, L)
                ibuf[0, rr, pl.ds(jj * L, L)] = fbuf[0, sl]
                obuf[0, rr, pl.ds(jj * L, L)] = vbuf[0, sl]
            pltpu.async_copy(obuf.at[0, rr], bias_sp.at[ibuf.at[0, rr]],
                             ssem0, add=True).wait()
        pltpu.sync_copy(bias_sp, bias_hbm)


def _build_w_bias(rows, cols, vals, bidx, bvals):
    nnz = rows.shape[0]
    grp = NS * CH * 2
    nnzp = ((nnz + grp - 1) // grp) * grp
    pad = nnzp + CH - nnz
    flat = rows * N + cols
    flat = jnp.concatenate([flat, jnp.full((pad,), 1 << 30, jnp.int32)])
    vals = jnp.concatenate([vals, jnp.zeros((pad,), jnp.float32)])
    zeros = jnp.zeros((RS_BIG,), jnp.float32)

    mesh = plsc.VectorSubcoreMesh(core_axis_name="c", subcore_axis_name="s")
    f = pl.kernel(
        _scatter_body,
        out_type=[jax.ShapeDtypeStruct((N * N,), jnp.float32),
                  jax.ShapeDtypeStruct((N,), jnp.float32)],
        mesh=mesh,
        scratch_types=[
            pltpu.VMEM_SHARED((R + NS * L,), jnp.float32),
            pltpu.VMEM_SHARED((N,), jnp.float32),
            pltpu.VMEM((2, CH), jnp.int32),
            pltpu.VMEM((2, CH), jnp.float32),
            pltpu.VMEM((2, GROUPS, GW), jnp.int32),
            pltpu.VMEM((2, GROUPS, GW), jnp.float32),
            pltpu.SemaphoreType.DMA,
            pltpu.SemaphoreType.DMA,
            pltpu.SemaphoreType.DMA,
            pltpu.SemaphoreType.DMA,
        ],
    )
    return f(flat, vals, bidx, bvals, zeros)


BN = 512
BK = 4096


def _matmul_body(x_ref, w_ref, b_ref, o_ref):
    k = pl.program_id(1)

    @pl.when(k == 0)
    def _():
        o_ref[...] = jnp.broadcast_to(b_ref[...], o_ref.shape)

    o_ref[...] += lax.dot_general(
        x_ref[...], w_ref[...],
        (((1,), (1,)), ((), ())),
        preferred_element_type=jnp.float32,
        precision=lax.Precision.DEFAULT,
    )


def _matmul(x, w, bias):
    grid = (N // BN, N // BK)
    return pl.pallas_call(
        _matmul_body,
        grid=grid,
        in_specs=[
            pl.BlockSpec((B, BK), lambda n, k: (0, k)),
            pl.BlockSpec((BN, BK), lambda n, k: (n, k)),
            pl.BlockSpec((1, BN), lambda n, k: (0, n)),
        ],
        out_specs=pl.BlockSpec((B, BN), lambda n, k: (0, n)),
        out_shape=jax.ShapeDtypeStruct((B, N), jnp.float32),
        compiler_params=pltpu.CompilerParams(
            dimension_semantics=("parallel", "arbitrary")),
    )(x, w, bias)


@jax.jit
def kernel(input, weight_rows, weight_cols, weight_vals, bias_idx, bias_vals):
    w_flat, bias = _build_w_bias(weight_rows.astype(jnp.int32),
                                 weight_cols.astype(jnp.int32),
                                 weight_vals, bias_idx.astype(jnp.int32),
                                 bias_vals)
    w = w_flat.reshape(N, N)
    return _matmul(input, w, bias.reshape(1, N))
